# Initial kernel scaffold; baseline (speedup 1.0000x reference)
import jax
import jax.numpy as jnp
from jax.experimental import pallas as pl


def kernel(input_ids, emb_table, fc_w, fc_b):
    raise NotImplementedError("write your pallas kernel here")


# speedup vs baseline: 3.3802x; 3.3802x over previous
import jax
import jax.numpy as jnp
from jax import lax
from jax.experimental import pallas as pl
from jax.experimental.pallas import tpu as pltpu
from jax.experimental.pallas import tpu_sc as plsc

VOCAB_SZ = 100000
EMBED = 128
NCLS = 64
BATCH_SZ = 4096
SEQ_LEN = 200
SEQ_PAD = 208
CHUNK = 104
LANES = 16
NCHUNK = NCLS // LANES

NC, NS = 2, 16
NW = NC * NS
BPW = BATCH_SZ // NW
IDX_ROWS_PW = BPW * SEQ_PAD // CHUNK

VBLK = 1000


def _proj_body(emb_ref, w_ref, out_ref):
    out_ref[...] = lax.dot_general(
        emb_ref[...], w_ref[...],
        dimension_numbers=(((1,), (1,)), ((), ())),
        preferred_element_type=jnp.float32)


def _pool_body(ids_hbm, proj_hbm, fcb_hbm, out_hbm, idx_v, buf_v, acc_v,
               bias_v, sem):
    wid = lax.axis_index("s") * NC + lax.axis_index("c")
    pltpu.sync_copy(ids_hbm.at[pl.ds(wid * IDX_ROWS_PW, IDX_ROWS_PW)], idx_v)
    pltpu.sync_copy(fcb_hbm, bias_v)

    def batch_body(i, carry):
        pltpu.async_copy(proj_hbm.at[idx_v.at[2 * i]], buf_v.at[0], sem).wait()
        pltpu.async_copy(proj_hbm.at[idx_v.at[2 * i + 1]], buf_v.at[1],
                         sem).wait()

        def red_body(r, accs):
            new = []
            for c in range(NCHUNK):
                a = accs[c]
                a = a + buf_v[0, r, pl.ds(c * LANES, LANES)]
                a = a + buf_v[1, r, pl.ds(c * LANES, LANES)]
                new.append(a)
            return tuple(new)

        accs = tuple(bias_v[pl.ds(c * LANES, LANES)] * float(SEQ_LEN)
                     for c in range(NCHUNK))
        accs = lax.fori_loop(0, CHUNK, red_body, accs)
        for c in range(NCHUNK):
            acc_v[i, pl.ds(c * LANES, LANES)] = accs[c]
        return carry

    lax.fori_loop(0, BPW, batch_body, 0)
    pltpu.sync_copy(acc_v, out_hbm.at[pl.ds(wid * BPW, BPW)])


def kernel(input_ids, emb_table, fc_w, fc_b):
    proj = pl.pallas_call(
        _proj_body,
        grid=(VOCAB_SZ // VBLK,),
        in_specs=[
            pl.BlockSpec((VBLK, EMBED), lambda i: (i, 0)),
            pl.BlockSpec((NCLS, EMBED), lambda i: (0, 0)),
        ],
        out_specs=pl.BlockSpec((VBLK, NCLS), lambda i: (i, 0)),
        out_shape=jax.ShapeDtypeStruct((VOCAB_SZ, NCLS), jnp.float32),
    )(emb_table, fc_w)

    ids = jnp.pad(input_ids, ((0, 0), (0, SEQ_PAD - SEQ_LEN)))
    ids2 = ids.reshape(BATCH_SZ * SEQ_PAD // CHUNK, CHUNK)

    pool = pl.kernel(
        _pool_body,
        out_type=jax.ShapeDtypeStruct((BATCH_SZ, NCLS), jnp.float32),
        mesh=plsc.VectorSubcoreMesh(core_axis_name="c", subcore_axis_name="s"),
        compiler_params=pltpu.CompilerParams(use_tc_tiling_on_sc=False),
        scratch_types=[
            pltpu.VMEM((IDX_ROWS_PW, CHUNK), jnp.int32),
            pltpu.VMEM((2, CHUNK, NCLS), jnp.float32),
            pltpu.VMEM((BPW, NCLS), jnp.float32),
            pltpu.VMEM((NCLS,), jnp.float32),
            pltpu.SemaphoreType.DMA,
        ],
    )
    return pool(ids2, proj, fc_b)


# speedup vs baseline: 13.1147x; 3.8798x over previous
import jax
import jax.numpy as jnp
from jax import lax
from jax.experimental import pallas as pl
from jax.experimental.pallas import tpu as pltpu
from jax.experimental.pallas import tpu_sc as plsc

VOCAB_SZ = 100000
EMBED = 128
NCLS = 64
BATCH_SZ = 4096
SEQ_LEN = 200
LANES = 16
NCHUNK = NCLS // LANES

NC, NS = 2, 16
NW = NC * NS
BPW = BATCH_SZ // NW
NSLOT = 8
NGRP = SEQ_LEN // NSLOT

VBLK = 1000


def _proj_body(emb_ref, w_ref, out_ref):
    out_ref[...] = lax.dot_general(
        emb_ref[...], w_ref[...],
        dimension_numbers=(((1,), (1,)), ((), ())),
        preferred_element_type=jnp.float32)


def _pool_body(ids_hbm, proj_hbm, fcb_hbm, out_hbm, idx_v, buf_v, bias_v,
               *sems):
    wid = lax.axis_index("s") * NC + lax.axis_index("c")
    pltpu.sync_copy(ids_hbm.at[pl.ds(wid * SEQ_LEN, SEQ_LEN)], idx_v)
    pltpu.sync_copy(fcb_hbm, bias_v)

    for k in range(NSLOT):
        pltpu.async_copy(proj_hbm.at[idx_v.at[k]], buf_v.at[k], sems[k])

    def grp_body(g, carry):
        for k in range(NSLOT):
            pltpu.make_async_copy(proj_hbm.at[pl.ds(0, BPW)], buf_v.at[k],
                                  sems[k]).wait()
            pltpu.async_copy(proj_hbm.at[idx_v.at[g * NSLOT + k]],
                             buf_v.at[k], sems[k], add=True)
        return carry

    lax.fori_loop(1, NGRP, grp_body, 0)
    for k in range(NSLOT):
        pltpu.make_async_copy(proj_hbm.at[pl.ds(0, BPW)], buf_v.at[k],
                              sems[k]).wait()

    def comb_body(b, carry):
        for c in range(NCHUNK):
            a = bias_v[pl.ds(c * LANES, LANES)] * float(SEQ_LEN)
            for k in range(NSLOT):
                a = a + buf_v[k, b, pl.ds(c * LANES, LANES)]
            buf_v[0, b, pl.ds(c * LANES, LANES)] = a
        return carry

    lax.fori_loop(0, BPW, comb_body, 0)
    pltpu.sync_copy(buf_v.at[0], out_hbm.at[pl.ds(wid * BPW, BPW)])


def kernel(input_ids, emb_table, fc_w, fc_b):
    proj = pl.pallas_call(
        _proj_body,
        grid=(VOCAB_SZ // VBLK,),
        in_specs=[
            pl.BlockSpec((VBLK, EMBED), lambda i: (i, 0)),
            pl.BlockSpec((NCLS, EMBED), lambda i: (0, 0)),
        ],
        out_specs=pl.BlockSpec((VBLK, NCLS), lambda i: (i, 0)),
        out_shape=jax.ShapeDtypeStruct((VOCAB_SZ, NCLS), jnp.float32),
    )(emb_table, fc_w)

    ids_t = (input_ids.reshape(NW, BPW, SEQ_LEN)
             .transpose(0, 2, 1)
             .reshape(NW * SEQ_LEN, BPW))

    pool = pl.kernel(
        _pool_body,
        out_type=jax.ShapeDtypeStruct((BATCH_SZ, NCLS), jnp.float32),
        mesh=plsc.VectorSubcoreMesh(core_axis_name="c", subcore_axis_name="s"),
        compiler_params=pltpu.CompilerParams(use_tc_tiling_on_sc=False),
        scratch_types=[
            pltpu.VMEM((SEQ_LEN, BPW), jnp.int32),
            pltpu.VMEM((NSLOT, BPW, NCLS), jnp.float32),
            pltpu.VMEM((NCLS,), jnp.float32),
        ] + [pltpu.SemaphoreType.DMA] * NSLOT,
    )
    return pool(ids_t, proj, fc_b)
